# baseline (speedup 1.0000x reference)
import jax
import jax.numpy as jnp
from jax.experimental import pallas as pl

_B, _S, _D = 1, 2048, 2048
_H, _HKV, _HD = 16, 4, 128
_G = _H // _HKV
_OP = _H * _HD + 2 * (_HKV * _HD)
_QP = _H * _HD
_KP = _HKV * _HD

_BS = 256
_BQ = 256
_BK = 256
_NQ = _S // _BQ
_NK = _S // _BK
_SCALE = _HD ** -0.5


def _qkv_rope_kernel(x_ref, w_ref, cos_ref, sin_ref, q_ref, k_ref, v_ref):
    x = x_ref[...]
    w = w_ref[...]
    qkv = jax.lax.dot_general(
        x, w, (((1,), (1,)), ((), ())),
        preferred_element_type=jnp.float32)
    cos = cos_ref[...]
    sin = sin_ref[...]
    c = cos[:, None, :]
    s = sin[:, None, :]

    def rope(t):
        t1 = t[..., :_HD // 2]
        t2 = t[..., _HD // 2:]
        rot = jnp.concatenate([-t2, t1], axis=-1)
        return t * c + rot * s

    q = qkv[:, :_QP].reshape(_BS, _H, _HD)
    k = qkv[:, _QP:_QP + _KP].reshape(_BS, _HKV, _HD)
    v = qkv[:, _QP + _KP:]
    q = rope(q).reshape(_BS, _QP)
    k = rope(k).reshape(_BS, _KP)
    q_ref[...] = q.astype(jnp.bfloat16)
    k_ref[...] = k.astype(jnp.bfloat16)
    v_ref[...] = v.astype(jnp.bfloat16)


def _attn_kernel(q_ref, k_ref, v_ref, o_ref):
    i = pl.program_id(1)
    q = q_ref[...]
    row = i * _BQ + jax.lax.broadcasted_iota(jnp.int32, (_BQ, _BK), 0)

    def body(j, carry):
        m, l, acc = carry
        k = k_ref[pl.ds(j * _BK, _BK), :]
        v = v_ref[pl.ds(j * _BK, _BK), :]
        sc = jax.lax.dot_general(
            q, k, (((1,), (1,)), ((), ())),
            preferred_element_type=jnp.float32) * _SCALE
        col = j * _BK + jax.lax.broadcasted_iota(jnp.int32, (_BQ, _BK), 1)
        sc = jnp.where(col <= row, sc, -1e30)
        m_new = jnp.maximum(m, jnp.max(sc, axis=-1, keepdims=True))
        p = jnp.exp(sc - m_new)
        alpha = jnp.exp(m - m_new)
        l_new = alpha * l + jnp.sum(p, axis=-1, keepdims=True)
        pv = jax.lax.dot_general(
            p.astype(jnp.bfloat16), v, (((1,), (0,)), ((), ())),
            preferred_element_type=jnp.float32)
        return m_new, l_new, acc * alpha + pv

    m0 = jnp.full((_BQ, 1), -jnp.inf, dtype=jnp.float32)
    l0 = jnp.zeros((_BQ, 1), dtype=jnp.float32)
    a0 = jnp.zeros((_BQ, _HD), dtype=jnp.float32)
    m, l, acc = jax.lax.fori_loop(0, i + 1, body, (m0, l0, a0))
    o_ref[...] = (acc / l).astype(jnp.bfloat16)


def _oproj_kernel(a_ref, w_ref, o_ref):
    o_ref[...] = jax.lax.dot_general(
        a_ref[...], w_ref[...], (((1,), (1,)), ((), ())),
        preferred_element_type=jnp.float32)


def kernel(hidden_states, cos, sin, Wqkv, Wo):
    x = hidden_states[0].astype(jnp.bfloat16)
    wqkv = Wqkv.astype(jnp.bfloat16)
    wo = Wo.astype(jnp.bfloat16)
    cos2 = cos[0]
    sin2 = sin[0]

    q, k, v = pl.pallas_call(
        _qkv_rope_kernel,
        grid=(_NQ,),
        in_specs=[
            pl.BlockSpec((_BS, _D), lambda i: (i, 0)),
            pl.BlockSpec((_OP, _D), lambda i: (0, 0)),
            pl.BlockSpec((_BS, _HD), lambda i: (i, 0)),
            pl.BlockSpec((_BS, _HD), lambda i: (i, 0)),
        ],
        out_specs=[
            pl.BlockSpec((_BS, _QP), lambda i: (i, 0)),
            pl.BlockSpec((_BS, _KP), lambda i: (i, 0)),
            pl.BlockSpec((_BS, _KP), lambda i: (i, 0)),
        ],
        out_shape=[
            jax.ShapeDtypeStruct((_S, _QP), jnp.bfloat16),
            jax.ShapeDtypeStruct((_S, _KP), jnp.bfloat16),
            jax.ShapeDtypeStruct((_S, _KP), jnp.bfloat16),
        ],
    )(x, wqkv, cos2, sin2)

    attn = pl.pallas_call(
        _attn_kernel,
        grid=(_H, _NQ),
        in_specs=[
            pl.BlockSpec((_BQ, _HD), lambda h, i: (i, h)),
            pl.BlockSpec((_S, _HD), lambda h, i: (0, h // _G)),
            pl.BlockSpec((_S, _HD), lambda h, i: (0, h // _G)),
        ],
        out_specs=pl.BlockSpec((_BQ, _HD), lambda h, i: (i, h)),
        out_shape=jax.ShapeDtypeStruct((_S, _QP), jnp.bfloat16),
    )(q, k, v)

    out = pl.pallas_call(
        _oproj_kernel,
        grid=(_NQ,),
        in_specs=[
            pl.BlockSpec((_BS, _QP), lambda i: (i, 0)),
            pl.BlockSpec((_D, _QP), lambda i: (0, 0)),
        ],
        out_specs=pl.BlockSpec((_BS, _D), lambda i: (i, 0)),
        out_shape=jax.ShapeDtypeStruct((_S, _D), jnp.float32),
    )(attn, wo)

    return out[None]
